# baseline (speedup 1.0000x reference)
import functools

import jax
import jax.numpy as jnp
from jax import lax
from jax.experimental import pallas as pl
from jax.experimental.pallas import tpu as pltpu
from jax.experimental.pallas import tpu_sc as plsc

N_NODES = 10000
E_EDGES = 320000
D = 128
DOUT = 2
DPAD = 16
NC, NS = 2, 16
NW = NC * NS
B = 128
NB = 4
K = 80
E_PAD = NW * B * K
CW = 64
K2 = 160
E_PAD2 = NS * B * K2
NP = 10240
RPT = NP // NS
BLK = 1024

_mesh = plsc.VectorSubcoreMesh(
    core_axis_name="c", subcore_axis_name="s", num_cores=NC, num_subcores=NS)


def _fill_rows(ref, rows, width, value):
  vec = jnp.full((16,), value, jnp.float32)
  def body(i, _):
    for c in range(width // 16):
      ref[i, pl.ds(c * 16, 16)] = vec
    return 0
  lax.fori_loop(0, rows, body, 0, unroll=False)


@functools.partial(
    pl.kernel,
    out_type=jax.ShapeDtypeStruct((NC, NP, DPAD), jnp.float32),
    mesh=_mesh,
    scratch_types=[
        pltpu.VMEM((B, DPAD), jnp.float32),
        pltpu.VMEM((B, DPAD), jnp.float32),
        pltpu.VMEM((K, B), jnp.int32),
        pltpu.VMEM_SHARED((NP, DPAD), jnp.float32),
    ],
    compiler_params=pltpu.CompilerParams(use_tc_tiling_on_sc=False),
)
def _deg_kernel(dst_hbm, out_hbm, ones_v, zeros_v, idx_v, acc_sh):
  cid = lax.axis_index("c")
  sid = lax.axis_index("s")
  wid = cid * NS + sid
  _fill_rows(ones_v, B, DPAD, 1.0)
  _fill_rows(zeros_v, B, DPAD, 0.0)

  def zrow(t, _):
    pltpu.sync_copy(zeros_v, acc_sh.at[pl.ds(sid * RPT + t * B, B)])
    return 0
  lax.fori_loop(0, RPT // B, zrow, 0)
  pltpu.sync_copy(dst_hbm.at[wid], idx_v)
  plsc.subcore_barrier()

  def chunk(j, _):
    pltpu.sync_copy(ones_v, acc_sh.at[idx_v.at[j]], add=True)
    return 0
  lax.fori_loop(0, K, chunk, 0)
  plsc.subcore_barrier()

  def wrow(t, _):
    r = sid * RPT + t * B
    pltpu.sync_copy(acc_sh.at[pl.ds(r, B)], out_hbm.at[cid, pl.ds(r, B)])
    return 0
  lax.fori_loop(0, RPT // B, wrow, 0)


@functools.partial(
    pl.kernel,
    out_type=jax.ShapeDtypeStruct((NC, NP, CW), jnp.float32),
    mesh=_mesh,
    scratch_types=[
        pltpu.VMEM((B, CW), jnp.float32),
        pltpu.VMEM((K2, B), jnp.int32),
        pltpu.VMEM((K2, B), jnp.int32),
        [pltpu.VMEM((B, CW), jnp.float32)] * 2,
        [pltpu.SemaphoreType.DMA] * 2,
        pltpu.VMEM_SHARED((NP, CW), jnp.float32),
    ],
    compiler_params=pltpu.CompilerParams(use_tc_tiling_on_sc=False),
)
def _agg128(h_hbm, src_hbm, dst_hbm, out_hbm, zeros_v, src_v, dst_v, rows,
            semg, acc_sh):
  cid = lax.axis_index("c")
  sid = lax.axis_index("s")
  _fill_rows(zeros_v, B, CW, 0.0)

  def zrow(t, _):
    pltpu.sync_copy(zeros_v, acc_sh.at[pl.ds(sid * RPT + t * B, B)])
    return 0
  lax.fori_loop(0, RPT // B, zrow, 0)
  pltpu.sync_copy(src_hbm.at[sid], src_v)
  pltpu.sync_copy(dst_hbm.at[sid], dst_v)
  plsc.subcore_barrier()

  table = h_hbm.at[cid]
  def chunk(j, _):
    pltpu.async_copy(table.at[src_v.at[j]], rows[0], semg[0]).wait()
    pltpu.sync_copy(rows[0], acc_sh.at[dst_v.at[j]], add=True)
    return 0
  lax.fori_loop(0, K2, chunk, 0)
  plsc.subcore_barrier()

  def wrow(t, _):
    r = sid * RPT + t * B
    pltpu.sync_copy(acc_sh.at[pl.ds(r, B)], out_hbm.at[cid, pl.ds(r, B)])
    return 0
  lax.fori_loop(0, RPT // B, wrow, 0)


@functools.partial(
    pl.kernel,
    out_type=jax.ShapeDtypeStruct((NC, NP, DPAD), jnp.float32),
    mesh=_mesh,
    scratch_types=[
        pltpu.VMEM((B, DPAD), jnp.float32),
        pltpu.VMEM((K, B), jnp.int32),
        pltpu.VMEM((K, B), jnp.int32),
        [pltpu.VMEM((B, DPAD), jnp.float32)] * NB,
        [pltpu.SemaphoreType.DMA] * NB,
        [pltpu.SemaphoreType.DMA] * NB,
        pltpu.VMEM_SHARED((NP, DPAD), jnp.float32),
    ],
    compiler_params=pltpu.CompilerParams(use_tc_tiling_on_sc=False),
)
def _agg16(h_hbm, src_hbm, dst_hbm, out_hbm, zeros_v, src_v, dst_v, rows,
           semg, sems, acc_sh):
  cid = lax.axis_index("c")
  sid = lax.axis_index("s")
  wid = cid * NS + sid
  _fill_rows(zeros_v, B, DPAD, 0.0)

  def zrow(t, _):
    pltpu.sync_copy(zeros_v, acc_sh.at[pl.ds(sid * RPT + t * B, B)])
    return 0
  lax.fori_loop(0, RPT // B, zrow, 0)
  pltpu.sync_copy(src_hbm.at[wid], src_v)
  pltpu.sync_copy(dst_hbm.at[wid], dst_v)
  plsc.subcore_barrier()

  def gat(j, b):
    pltpu.async_copy(h_hbm.at[src_v.at[j]], rows[b], semg[b])
  def gat_wait(j, b):
    pltpu.make_async_copy(h_hbm.at[src_v.at[j]], rows[b], semg[b]).wait()
  def sca(j, b):
    pltpu.async_copy(rows[b], acc_sh.at[dst_v.at[j]], sems[b], add=True)
  def sca_wait(j, b):
    pltpu.make_async_copy(rows[b], acc_sh.at[dst_v.at[j]], sems[b]).wait()
  for b in range(NB):
    gat(b, b)
  def group(g, _):
    base = g * NB
    for b in range(NB):
      gat_wait(base + b, b)
      sca(base + b, b)
    for b in range(NB):
      sca_wait(base + b, b)
      gat(base + NB + b, b)
    return 0
  lax.fori_loop(0, K // NB - 1, group, 0)
  last = K - NB
  for b in range(NB):
    gat_wait(last + b, b)
    sca(last + b, b)
  for b in range(NB):
    sca_wait(last + b, b)
  plsc.subcore_barrier()

  def wrow(t, _):
    r = sid * RPT + t * B
    pltpu.sync_copy(acc_sh.at[pl.ds(r, B)], out_hbm.at[cid, pl.ds(r, B)])
    return 0
  lax.fori_loop(0, RPT // B, wrow, 0)


def _dinv_from(degp0, degp1):
  deg = degp0 + degp1 + 1.0
  return lax.rsqrt(deg)[:, 0:1]


def _mm1_body(x_ref, w_ref, degp_ref, o_ref):
  dinv = _dinv_from(degp_ref[0], degp_ref[1])
  h = jnp.dot(x_ref[...], w_ref[0], preferred_element_type=jnp.float32,
              precision=lax.Precision.HIGHEST)
  o_ref[0] = h * dinv


def _mm2_body(acc1_ref, hp_ref, degp_ref, w2_ref, b1_ref, o_ref):
  dinv = _dinv_from(degp_ref[0], degp_ref[1])
  acc = jnp.concatenate([acc1_ref[0] + hp_ref[0], acc1_ref[1] + hp_ref[1]],
                        axis=1)
  h1 = jnp.maximum(acc * dinv + b1_ref[...], 0.0)
  g = jnp.dot(h1 * dinv, w2_ref[...], preferred_element_type=jnp.float32,
              precision=lax.Precision.HIGHEST)
  row = pl.program_id(0) * BLK + lax.broadcasted_iota(jnp.int32, (BLK, DPAD), 0)
  o_ref[...] = jnp.where(row < N_NODES, g, 0.0)


def _final_body(a0a1_ref, g_ref, degp_ref, b2_ref, o_ref):
  dinv = _dinv_from(degp_ref[0], degp_ref[1])
  acc = a0a1_ref[0] + a0a1_ref[1] + g_ref[...]
  o_ref[...] = acc * dinv + b2_ref[...]


def kernel(x, edge_index, W1, b1, W2, b2):
  f32 = jnp.float32
  x_pad = jnp.pad(x, ((0, NP - N_NODES), (0, 0)))
  pad = jnp.full((E_PAD - E_EDGES,), N_NODES, jnp.int32)
  src3 = jnp.concatenate([edge_index[0], pad]).reshape(NW, K, B)
  dst3 = jnp.concatenate([edge_index[1], pad]).reshape(NW, K, B)
  pad2 = jnp.full((E_PAD2 - E_EDGES,), N_NODES, jnp.int32)
  src2 = jnp.concatenate([edge_index[0], pad2]).reshape(NS, K2, B)
  dst2 = jnp.concatenate([edge_index[1], pad2]).reshape(NS, K2, B)
  w1_split = W1.reshape(D, NC, CW).transpose(1, 0, 2)
  w2_pad = jnp.pad(W2, ((0, 0), (0, DPAD - DOUT)))
  b1r = b1.reshape(1, D)
  b2r = jnp.pad(b2, (0, DPAD - DOUT)).reshape(1, DPAD)

  degp = _deg_kernel(dst3)

  grid = (NP // BLK,)
  degp_spec = pl.BlockSpec((2, BLK, DPAD), lambda g: (0, g, 0))
  degp_spec2 = pl.BlockSpec((2, BLK, DPAD), lambda g, c: (0, g, 0))
  hprime = pl.pallas_call(
      _mm1_body,
      grid=(NP // BLK, NC),
      in_specs=[
          pl.BlockSpec((BLK, D), lambda g, c: (g, 0)),
          pl.BlockSpec((1, D, CW), lambda g, c: (c, 0, 0)),
          degp_spec2,
      ],
      out_specs=pl.BlockSpec((1, BLK, CW), lambda g, c: (c, g, 0)),
      out_shape=jax.ShapeDtypeStruct((NC, NP, CW), f32),
  )(x_pad, w1_split, degp)

  acc1 = _agg128(hprime, src2, dst2)

  g = pl.pallas_call(
      _mm2_body,
      grid=grid,
      in_specs=[
          pl.BlockSpec((2, BLK, CW), lambda g: (0, g, 0)),
          pl.BlockSpec((2, BLK, CW), lambda g: (0, g, 0)),
          degp_spec,
          pl.BlockSpec((D, DPAD), lambda g: (0, 0)),
          pl.BlockSpec((1, D), lambda g: (0, 0)),
      ],
      out_specs=pl.BlockSpec((BLK, DPAD), lambda g: (g, 0)),
      out_shape=jax.ShapeDtypeStruct((NP, DPAD), f32),
  )(acc1, hprime, degp, w2_pad, b1r)

  acc2 = _agg16(g, src3, dst3)

  out = pl.pallas_call(
      _final_body,
      grid=grid,
      in_specs=[
          pl.BlockSpec((2, BLK, DPAD), lambda g: (0, g, 0)),
          pl.BlockSpec((BLK, DPAD), lambda g: (g, 0)),
          degp_spec,
          pl.BlockSpec((1, DPAD), lambda g: (0, 0)),
      ],
      out_specs=pl.BlockSpec((BLK, DPAD), lambda g: (g, 0)),
      out_shape=jax.ShapeDtypeStruct((NP, DPAD), f32),
  )(acc2, g, degp, b2r)

  return out[:N_NODES, :DOUT]


# speedup vs baseline: 1.0001x; 1.0001x over previous
import functools

import jax
import jax.numpy as jnp
from jax import lax
from jax.experimental import pallas as pl
from jax.experimental.pallas import tpu as pltpu
from jax.experimental.pallas import tpu_sc as plsc

N_NODES = 10000
E_EDGES = 320000
D = 128
DOUT = 2
DPAD = 16
NC, NS = 2, 16
NW = NC * NS
B = 128
NB = 4
K = 80
E_PAD = NW * B * K
CW = 64
K2 = 160
E_PAD2 = NS * B * K2
NP = 10240
RPT = NP // NS
BLK = 1024

_mesh = plsc.VectorSubcoreMesh(
    core_axis_name="c", subcore_axis_name="s", num_cores=NC, num_subcores=NS)


def _fill_rows(ref, rows, width, value):
  vec = jnp.full((16,), value, jnp.float32)
  def body(i, _):
    for c in range(width // 16):
      ref[i, pl.ds(c * 16, 16)] = vec
    return 0
  lax.fori_loop(0, rows, body, 0, unroll=False)


@functools.partial(
    pl.kernel,
    out_type=jax.ShapeDtypeStruct((NC, NP, DPAD), jnp.float32),
    mesh=_mesh,
    scratch_types=[
        pltpu.VMEM((B, DPAD), jnp.float32),
        pltpu.VMEM((B, DPAD), jnp.float32),
        pltpu.VMEM((K, B), jnp.int32),
        pltpu.VMEM_SHARED((NP, DPAD), jnp.float32),
    ],
    compiler_params=pltpu.CompilerParams(use_tc_tiling_on_sc=False),
)
def _deg_kernel(dst_hbm, out_hbm, ones_v, zeros_v, idx_v, acc_sh):
  cid = lax.axis_index("c")
  sid = lax.axis_index("s")
  wid = cid * NS + sid
  _fill_rows(ones_v, B, DPAD, 1.0)
  _fill_rows(zeros_v, B, DPAD, 0.0)

  def zrow(t, _):
    pltpu.sync_copy(zeros_v, acc_sh.at[pl.ds(sid * RPT + t * B, B)])
    return 0
  lax.fori_loop(0, RPT // B, zrow, 0)
  pltpu.sync_copy(dst_hbm.at[wid], idx_v)
  plsc.subcore_barrier()

  def chunk(j, _):
    pltpu.sync_copy(ones_v, acc_sh.at[idx_v.at[j]], add=True)
    return 0
  lax.fori_loop(0, K, chunk, 0)
  plsc.subcore_barrier()

  def wrow(t, _):
    r = sid * RPT + t * B
    pltpu.sync_copy(acc_sh.at[pl.ds(r, B)], out_hbm.at[cid, pl.ds(r, B)])
    return 0
  lax.fori_loop(0, RPT // B, wrow, 0)


@functools.partial(
    pl.kernel,
    out_type=jax.ShapeDtypeStruct((NC, NP, CW), jnp.float32),
    mesh=_mesh,
    scratch_types=[
        pltpu.VMEM((B, CW), jnp.float32),
        pltpu.VMEM((K2, B), jnp.int32),
        pltpu.VMEM((K2, B), jnp.int32),
        pltpu.VMEM((B, CW), jnp.float32),
        pltpu.VMEM_SHARED((NP, CW), jnp.float32),
        pltpu.SemaphoreType.DMA,
    ],
    compiler_params=pltpu.CompilerParams(use_tc_tiling_on_sc=False),
)
def _agg128(h_hbm, src_hbm, dst_hbm, out_hbm, zeros_v, src_v, dst_v, rows_v,
            acc_sh, sem):
  cid = lax.axis_index("c")
  sid = lax.axis_index("s")
  _fill_rows(zeros_v, B, CW, 0.0)

  def zrow(t, _):
    pltpu.sync_copy(zeros_v, acc_sh.at[pl.ds(sid * RPT + t * B, B)])
    return 0
  lax.fori_loop(0, RPT // B, zrow, 0)
  pltpu.sync_copy(src_hbm.at[sid], src_v)
  pltpu.sync_copy(dst_hbm.at[sid], dst_v)
  plsc.subcore_barrier()

  def chunk(j, _):
    pltpu.async_copy(h_hbm.at[cid].at[src_v.at[j]], rows_v, sem).wait()
    pltpu.sync_copy(rows_v, acc_sh.at[dst_v.at[j]], add=True)
    return 0
  lax.fori_loop(0, K2, chunk, 0)
  plsc.subcore_barrier()

  def wrow(t, _):
    r = sid * RPT + t * B
    pltpu.sync_copy(acc_sh.at[pl.ds(r, B)], out_hbm.at[cid, pl.ds(r, B)])
    return 0
  lax.fori_loop(0, RPT // B, wrow, 0)


@functools.partial(
    pl.kernel,
    out_type=jax.ShapeDtypeStruct((NC, NP, DPAD), jnp.float32),
    mesh=_mesh,
    scratch_types=[
        pltpu.VMEM((B, DPAD), jnp.float32),
        pltpu.VMEM((K, B), jnp.int32),
        pltpu.VMEM((K, B), jnp.int32),
        [pltpu.VMEM((B, DPAD), jnp.float32)] * NB,
        [pltpu.SemaphoreType.DMA] * NB,
        [pltpu.SemaphoreType.DMA] * NB,
        pltpu.VMEM_SHARED((NP, DPAD), jnp.float32),
    ],
    compiler_params=pltpu.CompilerParams(use_tc_tiling_on_sc=False),
)
def _agg16(h_hbm, src_hbm, dst_hbm, out_hbm, zeros_v, src_v, dst_v, rows,
           semg, sems, acc_sh):
  cid = lax.axis_index("c")
  sid = lax.axis_index("s")
  wid = cid * NS + sid
  _fill_rows(zeros_v, B, DPAD, 0.0)

  def zrow(t, _):
    pltpu.sync_copy(zeros_v, acc_sh.at[pl.ds(sid * RPT + t * B, B)])
    return 0
  lax.fori_loop(0, RPT // B, zrow, 0)
  pltpu.sync_copy(src_hbm.at[wid], src_v)
  pltpu.sync_copy(dst_hbm.at[wid], dst_v)
  plsc.subcore_barrier()

  def gat(j, b):
    pltpu.async_copy(h_hbm.at[src_v.at[j]], rows[b], semg[b])
  def gat_wait(j, b):
    pltpu.make_async_copy(h_hbm.at[src_v.at[j]], rows[b], semg[b]).wait()
  def sca(j, b):
    pltpu.async_copy(rows[b], acc_sh.at[dst_v.at[j]], sems[b], add=True)
  def sca_wait(j, b):
    pltpu.make_async_copy(rows[b], acc_sh.at[dst_v.at[j]], sems[b]).wait()
  for b in range(NB):
    gat(b, b)
  def group(g, _):
    base = g * NB
    for b in range(NB):
      gat_wait(base + b, b)
      sca(base + b, b)
    for b in range(NB):
      sca_wait(base + b, b)
      gat(base + NB + b, b)
    return 0
  lax.fori_loop(0, K // NB - 1, group, 0)
  last = K - NB
  for b in range(NB):
    gat_wait(last + b, b)
    sca(last + b, b)
  for b in range(NB):
    sca_wait(last + b, b)
  plsc.subcore_barrier()

  def wrow(t, _):
    r = sid * RPT + t * B
    pltpu.sync_copy(acc_sh.at[pl.ds(r, B)], out_hbm.at[cid, pl.ds(r, B)])
    return 0
  lax.fori_loop(0, RPT // B, wrow, 0)


def _dinv_from(degp0, degp1):
  deg = degp0 + degp1 + 1.0
  return lax.rsqrt(deg)[:, 0:1]


def _mm1_body(x_ref, w_ref, degp_ref, o_ref):
  dinv = _dinv_from(degp_ref[0], degp_ref[1])
  h = jnp.dot(x_ref[...], w_ref[0], preferred_element_type=jnp.float32,
              precision=lax.Precision.HIGHEST)
  o_ref[0] = h * dinv


def _mm2_body(acc1_ref, hp_ref, degp_ref, w2_ref, b1_ref, o_ref):
  dinv = _dinv_from(degp_ref[0], degp_ref[1])
  acc = jnp.concatenate([acc1_ref[0] + hp_ref[0], acc1_ref[1] + hp_ref[1]],
                        axis=1)
  h1 = jnp.maximum(acc * dinv + b1_ref[...], 0.0)
  g = jnp.dot(h1 * dinv, w2_ref[...], preferred_element_type=jnp.float32,
              precision=lax.Precision.HIGHEST)
  row = pl.program_id(0) * BLK + lax.broadcasted_iota(jnp.int32, (BLK, DPAD), 0)
  o_ref[...] = jnp.where(row < N_NODES, g, 0.0)


def _final_body(a0a1_ref, g_ref, degp_ref, b2_ref, o_ref):
  dinv = _dinv_from(degp_ref[0], degp_ref[1])
  acc = a0a1_ref[0] + a0a1_ref[1] + g_ref[...]
  o_ref[...] = acc * dinv + b2_ref[...]


def kernel(x, edge_index, W1, b1, W2, b2):
  f32 = jnp.float32
  x_pad = jnp.pad(x, ((0, NP - N_NODES), (0, 0)))
  pad = jnp.full((E_PAD - E_EDGES,), N_NODES, jnp.int32)
  src3 = jnp.concatenate([edge_index[0], pad]).reshape(NW, K, B)
  dst3 = jnp.concatenate([edge_index[1], pad]).reshape(NW, K, B)
  pad2 = jnp.full((E_PAD2 - E_EDGES,), N_NODES, jnp.int32)
  src2 = jnp.concatenate([edge_index[0], pad2]).reshape(NS, K2, B)
  dst2 = jnp.concatenate([edge_index[1], pad2]).reshape(NS, K2, B)
  w1_split = W1.reshape(D, NC, CW).transpose(1, 0, 2)
  w2_pad = jnp.pad(W2, ((0, 0), (0, DPAD - DOUT)))
  b1r = b1.reshape(1, D)
  b2r = jnp.pad(b2, (0, DPAD - DOUT)).reshape(1, DPAD)

  degp = _deg_kernel(dst3)

  grid = (NP // BLK,)
  degp_spec = pl.BlockSpec((2, BLK, DPAD), lambda g: (0, g, 0))
  degp_spec2 = pl.BlockSpec((2, BLK, DPAD), lambda g, c: (0, g, 0))
  hprime = pl.pallas_call(
      _mm1_body,
      grid=(NP // BLK, NC),
      in_specs=[
          pl.BlockSpec((BLK, D), lambda g, c: (g, 0)),
          pl.BlockSpec((1, D, CW), lambda g, c: (c, 0, 0)),
          degp_spec2,
      ],
      out_specs=pl.BlockSpec((1, BLK, CW), lambda g, c: (c, g, 0)),
      out_shape=jax.ShapeDtypeStruct((NC, NP, CW), f32),
  )(x_pad, w1_split, degp)

  acc1 = _agg128(hprime, src2, dst2)

  g = pl.pallas_call(
      _mm2_body,
      grid=grid,
      in_specs=[
          pl.BlockSpec((2, BLK, CW), lambda g: (0, g, 0)),
          pl.BlockSpec((2, BLK, CW), lambda g: (0, g, 0)),
          degp_spec,
          pl.BlockSpec((D, DPAD), lambda g: (0, 0)),
          pl.BlockSpec((1, D), lambda g: (0, 0)),
      ],
      out_specs=pl.BlockSpec((BLK, DPAD), lambda g: (g, 0)),
      out_shape=jax.ShapeDtypeStruct((NP, DPAD), f32),
  )(acc1, hprime, degp, w2_pad, b1r)

  acc2 = _agg16(g, src3, dst3)

  out = pl.pallas_call(
      _final_body,
      grid=grid,
      in_specs=[
          pl.BlockSpec((2, BLK, DPAD), lambda g: (0, g, 0)),
          pl.BlockSpec((BLK, DPAD), lambda g: (g, 0)),
          degp_spec,
          pl.BlockSpec((1, DPAD), lambda g: (0, 0)),
      ],
      out_specs=pl.BlockSpec((BLK, DPAD), lambda g: (g, 0)),
      out_shape=jax.ShapeDtypeStruct((NP, DPAD), f32),
  )(acc2, g, degp, b2r)

  return out[:N_NODES, :DOUT]


# speedup vs baseline: 1.0628x; 1.0627x over previous
import functools

import jax
import jax.numpy as jnp
from jax import lax
from jax.experimental import pallas as pl
from jax.experimental.pallas import tpu as pltpu
from jax.experimental.pallas import tpu_sc as plsc

N_NODES = 10000
E_EDGES = 320000
D = 128
DOUT = 2
DPAD = 16
NC, NS = 2, 16
NW = NC * NS
B = 128
NB = 4
K = 80
E_PAD = NW * B * K
CW = 64
K2 = 160
E_PAD2 = NS * B * K2
NP = 10240
RPT = NP // NS
BLK = 1024

_mesh = plsc.VectorSubcoreMesh(
    core_axis_name="c", subcore_axis_name="s", num_cores=NC, num_subcores=NS)


def _fill_rows(ref, rows, width, value):
  vec = jnp.full((16,), value, jnp.float32)
  def body(i, _):
    for c in range(width // 16):
      ref[i, pl.ds(c * 16, 16)] = vec
    return 0
  lax.fori_loop(0, rows, body, 0, unroll=False)


@functools.partial(
    pl.kernel,
    out_type=jax.ShapeDtypeStruct((NC, NP, DPAD), jnp.float32),
    mesh=_mesh,
    scratch_types=[
        pltpu.VMEM((B, DPAD), jnp.float32),
        pltpu.VMEM((B, DPAD), jnp.float32),
        pltpu.VMEM((K, B), jnp.int32),
        pltpu.VMEM_SHARED((NP, DPAD), jnp.float32),
    ],
    compiler_params=pltpu.CompilerParams(use_tc_tiling_on_sc=False),
)
def _deg_kernel(dst_hbm, out_hbm, ones_v, zeros_v, idx_v, acc_sh):
  cid = lax.axis_index("c")
  sid = lax.axis_index("s")
  wid = cid * NS + sid
  _fill_rows(ones_v, B, DPAD, 1.0)
  _fill_rows(zeros_v, B, DPAD, 0.0)

  def zrow(t, _):
    pltpu.sync_copy(zeros_v, acc_sh.at[pl.ds(sid * RPT + t * B, B)])
    return 0
  lax.fori_loop(0, RPT // B, zrow, 0)
  pltpu.sync_copy(dst_hbm.at[wid], idx_v)
  plsc.subcore_barrier()

  def chunk(j, _):
    pltpu.sync_copy(ones_v, acc_sh.at[idx_v.at[j]], add=True)
    return 0
  lax.fori_loop(0, K, chunk, 0)
  plsc.subcore_barrier()

  def wrow(t, _):
    r = sid * RPT + t * B
    pltpu.sync_copy(acc_sh.at[pl.ds(r, B)], out_hbm.at[cid, pl.ds(r, B)])
    return 0
  lax.fori_loop(0, RPT // B, wrow, 0)


@functools.partial(
    pl.kernel,
    out_type=jax.ShapeDtypeStruct((NC, NP, CW), jnp.float32),
    mesh=_mesh,
    scratch_types=[
        pltpu.VMEM((B, CW), jnp.float32),
        pltpu.VMEM((K2, B), jnp.int32),
        pltpu.VMEM((K2, B), jnp.int32),
        pltpu.VMEM((B, CW), jnp.float32),
        pltpu.VMEM_SHARED((NP, CW), jnp.float32),
        pltpu.SemaphoreType.DMA,
    ],
    compiler_params=pltpu.CompilerParams(use_tc_tiling_on_sc=False),
)
def _agg128(h_hbm, src_hbm, dst_hbm, out_hbm, zeros_v, src_v, dst_v, rows_v,
            acc_sh, sem):
  cid = lax.axis_index("c")
  sid = lax.axis_index("s")
  _fill_rows(zeros_v, B, CW, 0.0)

  def zrow(t, _):
    pltpu.sync_copy(zeros_v, acc_sh.at[pl.ds(sid * RPT + t * B, B)])
    return 0
  lax.fori_loop(0, RPT // B, zrow, 0)
  pltpu.sync_copy(src_hbm.at[sid], src_v)
  pltpu.sync_copy(dst_hbm.at[sid], dst_v)
  plsc.subcore_barrier()

  def chunk(j, _):
    pltpu.async_copy(h_hbm.at[cid].at[src_v.at[j]], rows_v, sem).wait()
    pltpu.sync_copy(rows_v, acc_sh.at[dst_v.at[j]], add=True)
    return 0
  lax.fori_loop(0, K2, chunk, 0)
  plsc.subcore_barrier()

  def wrow(t, _):
    r = sid * RPT + t * B
    pltpu.sync_copy(acc_sh.at[pl.ds(r, B)], out_hbm.at[cid, pl.ds(r, B)])
    return 0
  lax.fori_loop(0, RPT // B, wrow, 0)


@functools.partial(
    pl.kernel,
    out_type=jax.ShapeDtypeStruct((NC, NP, DPAD), jnp.float32),
    mesh=_mesh,
    scratch_types=[
        pltpu.VMEM((B, DPAD), jnp.float32),
        pltpu.VMEM((K, B), jnp.int32),
        pltpu.VMEM((K, B), jnp.int32),
        [pltpu.VMEM((B, DPAD), jnp.float32)] * NB,
        [pltpu.SemaphoreType.DMA] * NB,
        [pltpu.SemaphoreType.DMA] * NB,
        pltpu.VMEM_SHARED((NP, DPAD), jnp.float32),
    ],
    compiler_params=pltpu.CompilerParams(use_tc_tiling_on_sc=False),
)
def _agg16(h_hbm, src_hbm, dst_hbm, out_hbm, zeros_v, src_v, dst_v, rows,
           semg, sems, acc_sh):
  cid = lax.axis_index("c")
  sid = lax.axis_index("s")
  wid = cid * NS + sid
  _fill_rows(zeros_v, B, DPAD, 0.0)

  def zrow(t, _):
    pltpu.sync_copy(zeros_v, acc_sh.at[pl.ds(sid * RPT + t * B, B)])
    return 0
  lax.fori_loop(0, RPT // B, zrow, 0)
  pltpu.sync_copy(src_hbm.at[wid], src_v)
  pltpu.sync_copy(dst_hbm.at[wid], dst_v)
  plsc.subcore_barrier()

  def gat(j, b):
    pltpu.async_copy(h_hbm.at[src_v.at[j]], rows[b], semg[b])
  def gat_wait(j, b):
    pltpu.make_async_copy(h_hbm.at[src_v.at[j]], rows[b], semg[b]).wait()
  def sca(j, b):
    pltpu.async_copy(rows[b], acc_sh.at[dst_v.at[j]], sems[b], add=True)
  def sca_wait(j, b):
    pltpu.make_async_copy(rows[b], acc_sh.at[dst_v.at[j]], sems[b]).wait()
  for b in range(NB):
    gat(b, b)
  def group(g, _):
    base = g * NB
    for b in range(NB):
      gat_wait(base + b, b)
      sca(base + b, b)
    for b in range(NB):
      sca_wait(base + b, b)
      gat(base + NB + b, b)
    return 0
  lax.fori_loop(0, K // NB - 1, group, 0)
  last = K - NB
  for b in range(NB):
    gat_wait(last + b, b)
    sca(last + b, b)
  for b in range(NB):
    sca_wait(last + b, b)
  plsc.subcore_barrier()

  def wrow(t, _):
    r = sid * RPT + t * B
    pltpu.sync_copy(acc_sh.at[pl.ds(r, B)], out_hbm.at[cid, pl.ds(r, B)])
    return 0
  lax.fori_loop(0, RPT // B, wrow, 0)


def _dinv_from(degp0, degp1):
  deg = degp0 + degp1 + 1.0
  return lax.rsqrt(deg)[:, 0:1]


def _mm1_body(x_ref, w_ref, degp_ref, o_ref):
  dinv = _dinv_from(degp_ref[0], degp_ref[1])
  h = jnp.dot(x_ref[...], w_ref[0], preferred_element_type=jnp.float32,
              precision=lax.Precision.HIGHEST)
  o_ref[0] = h * dinv


def _mm2_body(acc1_ref, hp_ref, degp_ref, w2_ref, b1_ref, o_ref):
  dinv = _dinv_from(degp_ref[0], degp_ref[1])
  acc = jnp.concatenate([acc1_ref[0] + hp_ref[0], acc1_ref[1] + hp_ref[1]],
                        axis=1)
  h1 = jnp.maximum(acc * dinv + b1_ref[...], 0.0)
  g = jnp.dot(h1 * dinv, w2_ref[...], preferred_element_type=jnp.float32,
              precision=lax.Precision.HIGHEST)
  row = pl.program_id(0) * BLK + lax.broadcasted_iota(jnp.int32, (BLK, DPAD), 0)
  o_ref[...] = jnp.where(row < N_NODES, g, 0.0)


def _final_body(a0a1_ref, g_ref, degp_ref, b2_ref, o_ref):
  dinv = _dinv_from(degp_ref[0], degp_ref[1])
  acc = a0a1_ref[0] + a0a1_ref[1] + g_ref[...]
  o_ref[...] = acc * dinv + b2_ref[...]


def kernel(x, edge_index, W1, b1, W2, b2):
  f32 = jnp.float32
  x_pad = jnp.pad(x, ((0, NP - N_NODES), (0, 0)))
  spread = N_NODES + jnp.arange(E_PAD - E_EDGES, dtype=jnp.int32) % (NP - N_NODES)
  pad = jnp.full((E_PAD - E_EDGES,), N_NODES, jnp.int32)
  src3 = jnp.concatenate([edge_index[0], pad]).reshape(NW, K, B)
  dst3 = jnp.concatenate([edge_index[1], spread]).reshape(NW, K, B)
  spread2 = N_NODES + jnp.arange(E_PAD2 - E_EDGES, dtype=jnp.int32) % (NP - N_NODES)
  pad2 = jnp.full((E_PAD2 - E_EDGES,), N_NODES, jnp.int32)
  src2 = jnp.concatenate([edge_index[0], pad2]).reshape(NS, K2, B)
  dst2 = jnp.concatenate([edge_index[1], spread2]).reshape(NS, K2, B)
  w1_split = W1.reshape(D, NC, CW).transpose(1, 0, 2)
  w2_pad = jnp.pad(W2, ((0, 0), (0, DPAD - DOUT)))
  b1r = b1.reshape(1, D)
  b2r = jnp.pad(b2, (0, DPAD - DOUT)).reshape(1, DPAD)

  degp = _deg_kernel(dst3)

  grid = (NP // BLK,)
  degp_spec = pl.BlockSpec((2, BLK, DPAD), lambda g: (0, g, 0))
  degp_spec2 = pl.BlockSpec((2, BLK, DPAD), lambda g, c: (0, g, 0))
  hprime = pl.pallas_call(
      _mm1_body,
      grid=(NP // BLK, NC),
      in_specs=[
          pl.BlockSpec((BLK, D), lambda g, c: (g, 0)),
          pl.BlockSpec((1, D, CW), lambda g, c: (c, 0, 0)),
          degp_spec2,
      ],
      out_specs=pl.BlockSpec((1, BLK, CW), lambda g, c: (c, g, 0)),
      out_shape=jax.ShapeDtypeStruct((NC, NP, CW), f32),
  )(x_pad, w1_split, degp)

  acc1 = _agg128(hprime, src2, dst2)

  g = pl.pallas_call(
      _mm2_body,
      grid=grid,
      in_specs=[
          pl.BlockSpec((2, BLK, CW), lambda g: (0, g, 0)),
          pl.BlockSpec((2, BLK, CW), lambda g: (0, g, 0)),
          degp_spec,
          pl.BlockSpec((D, DPAD), lambda g: (0, 0)),
          pl.BlockSpec((1, D), lambda g: (0, 0)),
      ],
      out_specs=pl.BlockSpec((BLK, DPAD), lambda g: (g, 0)),
      out_shape=jax.ShapeDtypeStruct((NP, DPAD), f32),
  )(acc1, hprime, degp, w2_pad, b1r)

  acc2 = _agg16(g, src3, dst3)

  out = pl.pallas_call(
      _final_body,
      grid=grid,
      in_specs=[
          pl.BlockSpec((2, BLK, DPAD), lambda g: (0, g, 0)),
          pl.BlockSpec((BLK, DPAD), lambda g: (g, 0)),
          degp_spec,
          pl.BlockSpec((1, DPAD), lambda g: (0, 0)),
      ],
      out_specs=pl.BlockSpec((BLK, DPAD), lambda g: (g, 0)),
      out_shape=jax.ShapeDtypeStruct((NP, DPAD), f32),
  )(acc2, g, degp, b2r)

  return out[:N_NODES, :DOUT]


# speedup vs baseline: 1.3758x; 1.2945x over previous
import functools

import jax
import jax.numpy as jnp
from jax import lax
from jax.experimental import pallas as pl
from jax.experimental.pallas import tpu as pltpu
from jax.experimental.pallas import tpu_sc as plsc

N_NODES = 10000
E_EDGES = 320000
D = 128
DOUT = 2
DPAD = 16
NC, NS = 2, 16
NW = NC * NS
B = 128
NB = 4
K = 79
E_PAD = NW * B * K
CW = 64
K2 = 157
E_PAD2 = NS * B * K2
NP = 10240
RPT = NP // NS
BLK = 1024

_mesh = plsc.VectorSubcoreMesh(
    core_axis_name="c", subcore_axis_name="s", num_cores=NC, num_subcores=NS)


def _fill_rows(ref, rows, width, value):
  vec = jnp.full((16,), value, jnp.float32)
  def body(i, _):
    for c in range(width // 16):
      ref[i, pl.ds(c * 16, 16)] = vec
    return 0
  lax.fori_loop(0, rows, body, 0, unroll=False)


@functools.partial(
    pl.kernel,
    out_type=jax.ShapeDtypeStruct((NC, NP, DPAD), jnp.float32),
    mesh=_mesh,
    scratch_types=[
        pltpu.VMEM((B, DPAD), jnp.float32),
        pltpu.VMEM((B, DPAD), jnp.float32),
        pltpu.VMEM((K, B), jnp.int32),
        pltpu.VMEM_SHARED((NP, DPAD), jnp.float32),
    ],
    compiler_params=pltpu.CompilerParams(use_tc_tiling_on_sc=False),
)
def _deg_kernel(dst_hbm, out_hbm, ones_v, zeros_v, idx_v, acc_sh):
  cid = lax.axis_index("c")
  sid = lax.axis_index("s")
  wid = cid * NS + sid
  _fill_rows(ones_v, B, DPAD, 1.0)
  _fill_rows(zeros_v, B, DPAD, 0.0)

  def zrow(t, _):
    pltpu.sync_copy(zeros_v, acc_sh.at[pl.ds(sid * RPT + t * B, B)])
    return 0
  lax.fori_loop(0, RPT // B, zrow, 0)
  pltpu.sync_copy(dst_hbm.at[wid], idx_v)
  plsc.subcore_barrier()

  def chunk(j, _):
    pltpu.sync_copy(ones_v, acc_sh.at[idx_v.at[j]], add=True)
    return 0
  lax.fori_loop(0, K, chunk, 0)
  plsc.subcore_barrier()

  def wrow(t, _):
    r = sid * RPT + t * B
    pltpu.sync_copy(acc_sh.at[pl.ds(r, B)], out_hbm.at[cid, pl.ds(r, B)])
    return 0
  lax.fori_loop(0, RPT // B, wrow, 0)


@functools.partial(
    pl.kernel,
    out_type=jax.ShapeDtypeStruct((NC, NP, CW), jnp.float32),
    mesh=_mesh,
    scratch_types=[
        pltpu.VMEM((B, CW), jnp.float32),
        pltpu.VMEM((K2, B), jnp.int32),
        pltpu.VMEM((K2, B), jnp.int32),
        pltpu.VMEM((B, CW), jnp.float32),
        pltpu.VMEM_SHARED((NP, CW), jnp.float32),
        pltpu.SemaphoreType.DMA,
    ],
    compiler_params=pltpu.CompilerParams(use_tc_tiling_on_sc=False),
)
def _agg128(h_hbm, src_hbm, dst_hbm, out_hbm, zeros_v, src_v, dst_v, rows_v,
            acc_sh, sem):
  cid = lax.axis_index("c")
  sid = lax.axis_index("s")
  _fill_rows(zeros_v, B, CW, 0.0)

  def zrow(t, _):
    pltpu.sync_copy(zeros_v, acc_sh.at[pl.ds(sid * RPT + t * B, B)])
    return 0
  lax.fori_loop(0, RPT // B, zrow, 0)
  pltpu.sync_copy(src_hbm.at[sid], src_v)
  pltpu.sync_copy(dst_hbm.at[sid], dst_v)
  plsc.subcore_barrier()

  def chunk(j, _):
    pltpu.async_copy(h_hbm.at[cid].at[src_v.at[j]], rows_v, sem).wait()
    pltpu.sync_copy(rows_v, acc_sh.at[dst_v.at[j]], add=True)
    return 0
  lax.fori_loop(0, K2, chunk, 0)
  plsc.subcore_barrier()

  def wrow(t, _):
    r = sid * RPT + t * B
    pltpu.sync_copy(acc_sh.at[pl.ds(r, B)], out_hbm.at[cid, pl.ds(r, B)])
    return 0
  lax.fori_loop(0, RPT // B, wrow, 0)


@functools.partial(
    pl.kernel,
    out_type=jax.ShapeDtypeStruct((NC, NP, DPAD), jnp.float32),
    mesh=_mesh,
    scratch_types=[
        pltpu.VMEM((B, DPAD), jnp.float32),
        pltpu.VMEM((K, B), jnp.int32),
        pltpu.VMEM((K, B), jnp.int32),
        [pltpu.VMEM((B, DPAD), jnp.float32)] * NB,
        [pltpu.SemaphoreType.DMA] * NB,
        [pltpu.SemaphoreType.DMA] * NB,
        pltpu.VMEM_SHARED((NP, DPAD), jnp.float32),
    ],
    compiler_params=pltpu.CompilerParams(use_tc_tiling_on_sc=False),
)
def _agg16(h_hbm, src_hbm, dst_hbm, out_hbm, zeros_v, src_v, dst_v, rows,
           semg, sems, acc_sh):
  cid = lax.axis_index("c")
  sid = lax.axis_index("s")
  wid = cid * NS + sid
  _fill_rows(zeros_v, B, DPAD, 0.0)

  def zrow(t, _):
    pltpu.sync_copy(zeros_v, acc_sh.at[pl.ds(sid * RPT + t * B, B)])
    return 0
  lax.fori_loop(0, RPT // B, zrow, 0)
  pltpu.sync_copy(src_hbm.at[wid], src_v)
  pltpu.sync_copy(dst_hbm.at[wid], dst_v)
  plsc.subcore_barrier()

  def gat(j, b):
    pltpu.async_copy(h_hbm.at[src_v.at[j]], rows[b], semg[b])
  def gat_wait(j, b):
    pltpu.make_async_copy(h_hbm.at[src_v.at[j]], rows[b], semg[b]).wait()
  def sca(j, b):
    pltpu.async_copy(rows[b], acc_sh.at[dst_v.at[j]], sems[b], add=True)
  def sca_wait(j, b):
    pltpu.make_async_copy(rows[b], acc_sh.at[dst_v.at[j]], sems[b]).wait()
  for b in range(NB):
    gat(b, b)
  def group(g, _):
    base = g * NB
    for b in range(NB):
      gat_wait(base + b, b)
      sca(base + b, b)
    for b in range(NB):
      sca_wait(base + b, b)
      gat(base + NB + b, b)
    return 0
  lax.fori_loop(0, K // NB - 1, group, 0)
  last = (K // NB - 1) * NB
  for b in range(NB):
    gat_wait(last + b, b)
    sca(last + b, b)
  for b in range(NB):
    sca_wait(last + b, b)
  for j in range(K // NB * NB, K):
    gat(j, 0)
    gat_wait(j, 0)
    sca(j, 0)
    sca_wait(j, 0)
  plsc.subcore_barrier()

  def wrow(t, _):
    r = sid * RPT + t * B
    pltpu.sync_copy(acc_sh.at[pl.ds(r, B)], out_hbm.at[cid, pl.ds(r, B)])
    return 0
  lax.fori_loop(0, RPT // B, wrow, 0)


def _dinv_from(degp0, degp1):
  deg = degp0 + degp1 + 1.0
  return lax.rsqrt(deg)[:, 0:1]


def _mm1_body(x_ref, w_ref, degp_ref, o_ref):
  dinv = _dinv_from(degp_ref[0], degp_ref[1])
  h = jnp.dot(x_ref[...], w_ref[0], preferred_element_type=jnp.float32,
              precision=lax.Precision.HIGHEST)
  o_ref[0] = h * dinv


def _mm2_body(acc1_ref, hp_ref, degp_ref, w2_ref, b1_ref, o_ref):
  dinv = _dinv_from(degp_ref[0], degp_ref[1])
  acc = jnp.concatenate([acc1_ref[0] + hp_ref[0], acc1_ref[1] + hp_ref[1]],
                        axis=1)
  h1 = jnp.maximum(acc * dinv + b1_ref[...], 0.0)
  g = jnp.dot(h1 * dinv, w2_ref[...], preferred_element_type=jnp.float32,
              precision=lax.Precision.HIGHEST)
  row = pl.program_id(0) * BLK + lax.broadcasted_iota(jnp.int32, (BLK, DPAD), 0)
  o_ref[...] = jnp.where(row < N_NODES, g, 0.0)


def _final_body(a0a1_ref, g_ref, degp_ref, b2_ref, o_ref):
  dinv = _dinv_from(degp_ref[0], degp_ref[1])
  acc = a0a1_ref[0] + a0a1_ref[1] + g_ref[...]
  o_ref[...] = acc * dinv + b2_ref[...]


def kernel(x, edge_index, W1, b1, W2, b2):
  f32 = jnp.float32
  x_pad = jnp.pad(x, ((0, NP - N_NODES), (0, 0)))
  spread = N_NODES + jnp.arange(E_PAD - E_EDGES, dtype=jnp.int32) % (NP - N_NODES)
  pad = jnp.full((E_PAD - E_EDGES,), N_NODES, jnp.int32)
  src3 = jnp.concatenate([edge_index[0], pad]).reshape(NW, K, B)
  dst3 = jnp.concatenate([edge_index[1], spread]).reshape(NW, K, B)
  spread2 = N_NODES + jnp.arange(E_PAD2 - E_EDGES, dtype=jnp.int32) % (NP - N_NODES)
  pad2 = jnp.full((E_PAD2 - E_EDGES,), N_NODES, jnp.int32)
  src2 = jnp.concatenate([edge_index[0], pad2]).reshape(NS, K2, B)
  dst2 = jnp.concatenate([edge_index[1], spread2]).reshape(NS, K2, B)
  w1_split = W1.reshape(D, NC, CW).transpose(1, 0, 2)
  w2_pad = jnp.pad(W2, ((0, 0), (0, DPAD - DOUT)))
  b1r = b1.reshape(1, D)
  b2r = jnp.pad(b2, (0, DPAD - DOUT)).reshape(1, DPAD)

  degp = _deg_kernel(dst3)

  grid = (NP // BLK,)
  degp_spec = pl.BlockSpec((2, BLK, DPAD), lambda g: (0, g, 0))
  degp_spec2 = pl.BlockSpec((2, BLK, DPAD), lambda g, c: (0, g, 0))
  hprime = pl.pallas_call(
      _mm1_body,
      grid=(NP // BLK, NC),
      in_specs=[
          pl.BlockSpec((BLK, D), lambda g, c: (g, 0)),
          pl.BlockSpec((1, D, CW), lambda g, c: (c, 0, 0)),
          degp_spec2,
      ],
      out_specs=pl.BlockSpec((1, BLK, CW), lambda g, c: (c, g, 0)),
      out_shape=jax.ShapeDtypeStruct((NC, NP, CW), f32),
  )(x_pad, w1_split, degp)

  acc1 = _agg128(hprime, src2, dst2)

  g = pl.pallas_call(
      _mm2_body,
      grid=grid,
      in_specs=[
          pl.BlockSpec((2, BLK, CW), lambda g: (0, g, 0)),
          pl.BlockSpec((2, BLK, CW), lambda g: (0, g, 0)),
          degp_spec,
          pl.BlockSpec((D, DPAD), lambda g: (0, 0)),
          pl.BlockSpec((1, D), lambda g: (0, 0)),
      ],
      out_specs=pl.BlockSpec((BLK, DPAD), lambda g: (g, 0)),
      out_shape=jax.ShapeDtypeStruct((NP, DPAD), f32),
  )(acc1, hprime, degp, w2_pad, b1r)

  acc2 = _agg16(g, src3, dst3)

  out = pl.pallas_call(
      _final_body,
      grid=grid,
      in_specs=[
          pl.BlockSpec((2, BLK, DPAD), lambda g: (0, g, 0)),
          pl.BlockSpec((BLK, DPAD), lambda g: (g, 0)),
          degp_spec,
          pl.BlockSpec((1, DPAD), lambda g: (0, 0)),
      ],
      out_specs=pl.BlockSpec((BLK, DPAD), lambda g: (g, 0)),
      out_shape=jax.ShapeDtypeStruct((NP, DPAD), f32),
  )(acc2, g, degp, b2r)

  return out[:N_NODES, :DOUT]


# speedup vs baseline: 1.7792x; 1.2932x over previous
import functools

import jax
import jax.numpy as jnp
from jax import lax
from jax.experimental import pallas as pl
from jax.experimental.pallas import tpu as pltpu
from jax.experimental.pallas import tpu_sc as plsc

N_NODES = 10000
E_EDGES = 320000
D = 128
DOUT = 2
DPAD = 16
NC, NS = 2, 16
NW = NC * NS
B = 128
NB = 4
K = 79
E_PAD = NW * B * K
CW = 64
K2 = 157
E_PAD2 = NS * B * K2
NP = 10240
RPT = NP // NS
BLK = 1024

_mesh = plsc.VectorSubcoreMesh(
    core_axis_name="c", subcore_axis_name="s", num_cores=NC, num_subcores=NS)


def _fill_rows(ref, rows, width, value):
  vec = jnp.full((16,), value, jnp.float32)
  def body(i, _):
    for c in range(width // 16):
      ref[i, pl.ds(c * 16, 16)] = vec
    return 0
  lax.fori_loop(0, rows, body, 0, unroll=False)


@functools.partial(
    pl.kernel,
    out_type=jax.ShapeDtypeStruct((NC, NP, DPAD), jnp.float32),
    mesh=_mesh,
    scratch_types=[
        pltpu.VMEM((B, DPAD), jnp.float32),
        pltpu.VMEM((B, DPAD), jnp.float32),
        pltpu.VMEM((K, B), jnp.int32),
        pltpu.VMEM_SHARED((NP, DPAD), jnp.float32),
    ],
    compiler_params=pltpu.CompilerParams(use_tc_tiling_on_sc=False),
)
def _deg_kernel(dst_hbm, out_hbm, ones_v, zeros_v, idx_v, acc_sh):
  cid = lax.axis_index("c")
  sid = lax.axis_index("s")
  wid = cid * NS + sid
  _fill_rows(ones_v, B, DPAD, 1.0)
  _fill_rows(zeros_v, B, DPAD, 0.0)

  def zrow(t, _):
    pltpu.sync_copy(zeros_v, acc_sh.at[pl.ds(sid * RPT + t * B, B)])
    return 0
  lax.fori_loop(0, RPT // B, zrow, 0)
  pltpu.sync_copy(dst_hbm.at[wid], idx_v)
  plsc.subcore_barrier()

  def chunk(j, _):
    pltpu.sync_copy(ones_v, acc_sh.at[idx_v.at[j]], add=True)
    return 0
  lax.fori_loop(0, K, chunk, 0)
  plsc.subcore_barrier()

  def wrow(t, _):
    r = sid * RPT + t * B
    pltpu.sync_copy(acc_sh.at[pl.ds(r, B)], out_hbm.at[cid, pl.ds(r, B)])
    return 0
  lax.fori_loop(0, RPT // B, wrow, 0)


@functools.partial(
    pl.kernel,
    out_type=jax.ShapeDtypeStruct((NC, NP, CW), jnp.float32),
    mesh=_mesh,
    scratch_types=[
        pltpu.VMEM((B, CW), jnp.float32),
        pltpu.VMEM((K2, B), jnp.int32),
        pltpu.VMEM((K2, B), jnp.int32),
        [pltpu.VMEM((B, CW), jnp.float32)] * 2,
        pltpu.VMEM_SHARED((NP, CW), jnp.float32),
        [pltpu.SemaphoreType.DMA] * 2,
    ],
    compiler_params=pltpu.CompilerParams(use_tc_tiling_on_sc=False),
)
def _agg128(h_hbm, src_hbm, dst_hbm, out_hbm, zeros_v, src_v, dst_v, rows,
            acc_sh, sem):
  cid = lax.axis_index("c")
  sid = lax.axis_index("s")
  _fill_rows(zeros_v, B, CW, 0.0)

  def zrow(t, _):
    pltpu.sync_copy(zeros_v, acc_sh.at[pl.ds(sid * RPT + t * B, B)])
    return 0
  lax.fori_loop(0, RPT // B, zrow, 0)
  pltpu.sync_copy(src_hbm.at[sid], src_v)
  pltpu.sync_copy(dst_hbm.at[sid], dst_v)
  plsc.subcore_barrier()

  def gat(j, b):
    pltpu.async_copy(h_hbm.at[cid].at[src_v.at[j]], rows[b], sem[b])
  def gat_wait(j, b):
    pltpu.make_async_copy(h_hbm.at[cid].at[src_v.at[j]], rows[b],
                          sem[b]).wait()
  def sca(j, b):
    pltpu.sync_copy(rows[b], acc_sh.at[dst_v.at[j]], add=True)
  gat(0, 0)
  def pair(g, _):
    gat(2 * g + 1, 1)
    gat_wait(2 * g, 0)
    sca(2 * g, 0)
    gat(2 * g + 2, 0)
    gat_wait(2 * g + 1, 1)
    sca(2 * g + 1, 1)
    return 0
  lax.fori_loop(0, (K2 - 1) // 2, pair, 0)
  j0 = ((K2 - 1) // 2) * 2
  for j in range(j0, K2):
    gat_wait(j, 0)
    sca(j, 0)
  plsc.subcore_barrier()

  def wrow(t, _):
    r = sid * RPT + t * B
    pltpu.sync_copy(acc_sh.at[pl.ds(r, B)], out_hbm.at[cid, pl.ds(r, B)])
    return 0
  lax.fori_loop(0, RPT // B, wrow, 0)


@functools.partial(
    pl.kernel,
    out_type=jax.ShapeDtypeStruct((NC, NP, DPAD), jnp.float32),
    mesh=_mesh,
    scratch_types=[
        pltpu.VMEM((B, DPAD), jnp.float32),
        pltpu.VMEM((K, B), jnp.int32),
        pltpu.VMEM((K, B), jnp.int32),
        [pltpu.VMEM((B, DPAD), jnp.float32)] * NB,
        [pltpu.SemaphoreType.DMA] * NB,
        [pltpu.SemaphoreType.DMA] * NB,
        pltpu.VMEM_SHARED((NP, DPAD), jnp.float32),
    ],
    compiler_params=pltpu.CompilerParams(use_tc_tiling_on_sc=False),
)
def _agg16(h_hbm, src_hbm, dst_hbm, out_hbm, zeros_v, src_v, dst_v, rows,
           semg, sems, acc_sh):
  cid = lax.axis_index("c")
  sid = lax.axis_index("s")
  wid = cid * NS + sid
  _fill_rows(zeros_v, B, DPAD, 0.0)

  def zrow(t, _):
    pltpu.sync_copy(zeros_v, acc_sh.at[pl.ds(sid * RPT + t * B, B)])
    return 0
  lax.fori_loop(0, RPT // B, zrow, 0)
  pltpu.sync_copy(src_hbm.at[wid], src_v)
  pltpu.sync_copy(dst_hbm.at[wid], dst_v)
  plsc.subcore_barrier()

  def gat(j, b):
    pltpu.async_copy(h_hbm.at[src_v.at[j]], rows[b], semg[b])
  def gat_wait(j, b):
    pltpu.make_async_copy(h_hbm.at[src_v.at[j]], rows[b], semg[b]).wait()
  def sca(j, b):
    pltpu.async_copy(rows[b], acc_sh.at[dst_v.at[j]], sems[b], add=True)
  def sca_wait(j, b):
    pltpu.make_async_copy(rows[b], acc_sh.at[dst_v.at[j]], sems[b]).wait()
  for b in range(NB):
    gat(b, b)
  def group(g, _):
    base = g * NB
    for b in range(NB):
      gat_wait(base + b, b)
      sca(base + b, b)
    for b in range(NB):
      sca_wait(base + b, b)
      gat(base + NB + b, b)
    return 0
  lax.fori_loop(0, K // NB - 1, group, 0)
  last = (K // NB - 1) * NB
  for b in range(NB):
    gat_wait(last + b, b)
    sca(last + b, b)
  for b in range(NB):
    sca_wait(last + b, b)
  for j in range(K // NB * NB, K):
    gat(j, 0)
    gat_wait(j, 0)
    sca(j, 0)
    sca_wait(j, 0)
  plsc.subcore_barrier()

  def wrow(t, _):
    r = sid * RPT + t * B
    pltpu.sync_copy(acc_sh.at[pl.ds(r, B)], out_hbm.at[cid, pl.ds(r, B)])
    return 0
  lax.fori_loop(0, RPT // B, wrow, 0)


def _dinv_from(degp0, degp1):
  deg = degp0 + degp1 + 1.0
  return lax.rsqrt(deg)[:, 0:1]


def _mm1_body(x_ref, w_ref, degp_ref, o_ref):
  dinv = _dinv_from(degp_ref[0], degp_ref[1])
  h = jnp.dot(x_ref[...], w_ref[0], preferred_element_type=jnp.float32)
  o_ref[0] = h * dinv


def _mm2_body(acc1_ref, hp_ref, degp_ref, w2_ref, b1_ref, o_ref):
  dinv = _dinv_from(degp_ref[0], degp_ref[1])
  acc = jnp.concatenate([acc1_ref[0] + hp_ref[0], acc1_ref[1] + hp_ref[1]],
                        axis=1)
  h1 = jnp.maximum(acc * dinv + b1_ref[...], 0.0)
  g = jnp.dot(h1 * dinv, w2_ref[...], preferred_element_type=jnp.float32)
  row = pl.program_id(0) * BLK + lax.broadcasted_iota(jnp.int32, (BLK, DPAD), 0)
  o_ref[...] = jnp.where(row < N_NODES, g, 0.0)


def _final_body(a0a1_ref, g_ref, degp_ref, b2_ref, o_ref):
  dinv = _dinv_from(degp_ref[0], degp_ref[1])
  acc = a0a1_ref[0] + a0a1_ref[1] + g_ref[...]
  o_ref[...] = acc * dinv + b2_ref[...]


def kernel(x, edge_index, W1, b1, W2, b2):
  f32 = jnp.float32
  x_pad = jnp.pad(x, ((0, NP - N_NODES), (0, 0)))
  spread = N_NODES + jnp.arange(E_PAD - E_EDGES, dtype=jnp.int32) % (NP - N_NODES)
  pad = jnp.full((E_PAD - E_EDGES,), N_NODES, jnp.int32)
  src3 = jnp.concatenate([edge_index[0], pad]).reshape(NW, K, B)
  dst3 = jnp.concatenate([edge_index[1], spread]).reshape(NW, K, B)
  spread2 = N_NODES + jnp.arange(E_PAD2 - E_EDGES, dtype=jnp.int32) % (NP - N_NODES)
  pad2 = jnp.full((E_PAD2 - E_EDGES,), N_NODES, jnp.int32)
  src2 = jnp.concatenate([edge_index[0], pad2]).reshape(NS, K2, B)
  dst2 = jnp.concatenate([edge_index[1], spread2]).reshape(NS, K2, B)
  w1_split = W1.reshape(D, NC, CW).transpose(1, 0, 2)
  w2_pad = jnp.pad(W2, ((0, 0), (0, DPAD - DOUT)))
  b1r = b1.reshape(1, D)
  b2r = jnp.pad(b2, (0, DPAD - DOUT)).reshape(1, DPAD)

  degp = _deg_kernel(dst3)

  grid = (NP // BLK,)
  degp_spec = pl.BlockSpec((2, BLK, DPAD), lambda g: (0, g, 0))
  degp_spec2 = pl.BlockSpec((2, BLK, DPAD), lambda g, c: (0, g, 0))
  hprime = pl.pallas_call(
      _mm1_body,
      grid=(NP // BLK, NC),
      in_specs=[
          pl.BlockSpec((BLK, D), lambda g, c: (g, 0)),
          pl.BlockSpec((1, D, CW), lambda g, c: (c, 0, 0)),
          degp_spec2,
      ],
      out_specs=pl.BlockSpec((1, BLK, CW), lambda g, c: (c, g, 0)),
      out_shape=jax.ShapeDtypeStruct((NC, NP, CW), f32),
  )(x_pad, w1_split, degp)

  acc1 = _agg128(hprime, src2, dst2)

  g = pl.pallas_call(
      _mm2_body,
      grid=grid,
      in_specs=[
          pl.BlockSpec((2, BLK, CW), lambda g: (0, g, 0)),
          pl.BlockSpec((2, BLK, CW), lambda g: (0, g, 0)),
          degp_spec,
          pl.BlockSpec((D, DPAD), lambda g: (0, 0)),
          pl.BlockSpec((1, D), lambda g: (0, 0)),
      ],
      out_specs=pl.BlockSpec((BLK, DPAD), lambda g: (g, 0)),
      out_shape=jax.ShapeDtypeStruct((NP, DPAD), f32),
  )(acc1, hprime, degp, w2_pad, b1r)

  acc2 = _agg16(g, src3, dst3)

  out = pl.pallas_call(
      _final_body,
      grid=grid,
      in_specs=[
          pl.BlockSpec((2, BLK, DPAD), lambda g: (0, g, 0)),
          pl.BlockSpec((BLK, DPAD), lambda g: (g, 0)),
          degp_spec,
          pl.BlockSpec((1, DPAD), lambda g: (0, 0)),
      ],
      out_specs=pl.BlockSpec((BLK, DPAD), lambda g: (g, 0)),
      out_shape=jax.ShapeDtypeStruct((NP, DPAD), f32),
  )(acc2, g, degp, b2r)

  return out[:N_NODES, :DOUT]


# speedup vs baseline: 1.8326x; 1.0300x over previous
import functools

import jax
import jax.numpy as jnp
from jax import lax
from jax.experimental import pallas as pl
from jax.experimental.pallas import tpu as pltpu
from jax.experimental.pallas import tpu_sc as plsc

N_NODES = 10000
E_EDGES = 320000
D = 128
DOUT = 2
DPAD = 16
NC, NS = 2, 16
NW = NC * NS
B = 128
NB = 4
K = 79
E_PAD = NW * B * K
CW = 64
K2 = 157
E_PAD2 = NS * B * K2
NP = 10240
RPT = NP // NS
BLK = 1024

_mesh = plsc.VectorSubcoreMesh(
    core_axis_name="c", subcore_axis_name="s", num_cores=NC, num_subcores=NS)


def _fill_rows(ref, rows, width, value):
  vec = jnp.full((16,), value, jnp.float32)
  def body(i, _):
    for c in range(width // 16):
      ref[i, pl.ds(c * 16, 16)] = vec
    return 0
  lax.fori_loop(0, rows, body, 0, unroll=False)


@functools.partial(
    pl.kernel,
    out_type=jax.ShapeDtypeStruct((NC, NP, DPAD), jnp.float32),
    mesh=_mesh,
    scratch_types=[
        pltpu.VMEM((B, DPAD), jnp.float32),
        pltpu.VMEM((B, DPAD), jnp.float32),
        pltpu.VMEM((K, B), jnp.int32),
        pltpu.VMEM_SHARED((NP, DPAD), jnp.float32),
    ],
    compiler_params=pltpu.CompilerParams(use_tc_tiling_on_sc=False),
)
def _deg_kernel(dst_hbm, out_hbm, ones_v, zeros_v, idx_v, acc_sh):
  cid = lax.axis_index("c")
  sid = lax.axis_index("s")
  wid = cid * NS + sid
  _fill_rows(ones_v, B, DPAD, 1.0)
  _fill_rows(zeros_v, B, DPAD, 0.0)

  def zrow(t, _):
    pltpu.sync_copy(zeros_v, acc_sh.at[pl.ds(sid * RPT + t * B, B)])
    return 0
  lax.fori_loop(0, RPT // B, zrow, 0)
  pltpu.sync_copy(dst_hbm.at[wid], idx_v)
  plsc.subcore_barrier()

  def chunk(j, _):
    pltpu.sync_copy(ones_v, acc_sh.at[idx_v.at[j]], add=True)
    return 0
  lax.fori_loop(0, K, chunk, 0)
  plsc.subcore_barrier()

  def wrow(t, _):
    r = sid * RPT + t * B
    pltpu.sync_copy(acc_sh.at[pl.ds(r, B)], out_hbm.at[cid, pl.ds(r, B)])
    return 0
  lax.fori_loop(0, RPT // B, wrow, 0)


@functools.partial(
    pl.kernel,
    out_type=jax.ShapeDtypeStruct((NC, NP, CW), jnp.float32),
    mesh=_mesh,
    scratch_types=[
        pltpu.VMEM((B, CW), jnp.float32),
        pltpu.VMEM((K2, B), jnp.int32),
        pltpu.VMEM((K2, B), jnp.int32),
        [pltpu.VMEM((B, CW), jnp.float32)] * NB,
        pltpu.VMEM_SHARED((NP, CW), jnp.float32),
        [pltpu.SemaphoreType.DMA] * NB,
        [pltpu.SemaphoreType.DMA] * NB,
    ],
    compiler_params=pltpu.CompilerParams(use_tc_tiling_on_sc=False),
)
def _agg128(h_hbm, src_hbm, dst_hbm, out_hbm, zeros_v, src_v, dst_v, rows,
            acc_sh, semg, sems):
  cid = lax.axis_index("c")
  sid = lax.axis_index("s")
  _fill_rows(zeros_v, B, CW, 0.0)

  def zrow(t, _):
    pltpu.sync_copy(zeros_v, acc_sh.at[pl.ds(sid * RPT + t * B, B)])
    return 0
  lax.fori_loop(0, RPT // B, zrow, 0)
  pltpu.sync_copy(src_hbm.at[sid], src_v)
  pltpu.sync_copy(dst_hbm.at[sid], dst_v)
  plsc.subcore_barrier()

  def gat(j, b):
    pltpu.async_copy(h_hbm.at[cid].at[src_v.at[j]], rows[b], semg[b])
  def gat_wait(j, b):
    pltpu.make_async_copy(h_hbm.at[cid].at[src_v.at[j]], rows[b],
                          semg[b]).wait()
  def sca(j, b):
    pltpu.async_copy(rows[b], acc_sh.at[dst_v.at[j]], sems[b], add=True)
  def sca_wait(j, b):
    pltpu.make_async_copy(rows[b], acc_sh.at[dst_v.at[j]], sems[b]).wait()
  for b in range(NB):
    gat(b, b)
  def group(g, _):
    base = g * NB
    for b in range(NB):
      gat_wait(base + b, b)
      sca(base + b, b)
    for b in range(NB):
      sca_wait(base + b, b)
      gat(base + NB + b, b)
    return 0
  lax.fori_loop(0, K2 // NB - 1, group, 0)
  last = (K2 // NB - 1) * NB
  for b in range(NB):
    gat_wait(last + b, b)
    sca(last + b, b)
  for b in range(NB):
    sca_wait(last + b, b)
  for j in range(K2 // NB * NB, K2):
    gat(j, 0)
    gat_wait(j, 0)
    sca(j, 0)
    sca_wait(j, 0)
  plsc.subcore_barrier()

  def wrow(t, _):
    r = sid * RPT + t * B
    pltpu.sync_copy(acc_sh.at[pl.ds(r, B)], out_hbm.at[cid, pl.ds(r, B)])
    return 0
  lax.fori_loop(0, RPT // B, wrow, 0)


@functools.partial(
    pl.kernel,
    out_type=jax.ShapeDtypeStruct((NC, NP, DPAD), jnp.float32),
    mesh=_mesh,
    scratch_types=[
        pltpu.VMEM((B, DPAD), jnp.float32),
        pltpu.VMEM((K, B), jnp.int32),
        pltpu.VMEM((K, B), jnp.int32),
        [pltpu.VMEM((B, DPAD), jnp.float32)] * NB,
        [pltpu.SemaphoreType.DMA] * NB,
        [pltpu.SemaphoreType.DMA] * NB,
        pltpu.VMEM_SHARED((NP, DPAD), jnp.float32),
    ],
    compiler_params=pltpu.CompilerParams(use_tc_tiling_on_sc=False),
)
def _agg16(h_hbm, src_hbm, dst_hbm, out_hbm, zeros_v, src_v, dst_v, rows,
           semg, sems, acc_sh):
  cid = lax.axis_index("c")
  sid = lax.axis_index("s")
  wid = cid * NS + sid
  _fill_rows(zeros_v, B, DPAD, 0.0)

  def zrow(t, _):
    pltpu.sync_copy(zeros_v, acc_sh.at[pl.ds(sid * RPT + t * B, B)])
    return 0
  lax.fori_loop(0, RPT // B, zrow, 0)
  pltpu.sync_copy(src_hbm.at[wid], src_v)
  pltpu.sync_copy(dst_hbm.at[wid], dst_v)
  plsc.subcore_barrier()

  def gat(j, b):
    pltpu.async_copy(h_hbm.at[src_v.at[j]], rows[b], semg[b])
  def gat_wait(j, b):
    pltpu.make_async_copy(h_hbm.at[src_v.at[j]], rows[b], semg[b]).wait()
  def sca(j, b):
    pltpu.async_copy(rows[b], acc_sh.at[dst_v.at[j]], sems[b], add=True)
  def sca_wait(j, b):
    pltpu.make_async_copy(rows[b], acc_sh.at[dst_v.at[j]], sems[b]).wait()
  for b in range(NB):
    gat(b, b)
  def group(g, _):
    base = g * NB
    for b in range(NB):
      gat_wait(base + b, b)
      sca(base + b, b)
    for b in range(NB):
      sca_wait(base + b, b)
      gat(base + NB + b, b)
    return 0
  lax.fori_loop(0, K // NB - 1, group, 0)
  last = (K // NB - 1) * NB
  for b in range(NB):
    gat_wait(last + b, b)
    sca(last + b, b)
  for b in range(NB):
    sca_wait(last + b, b)
  for j in range(K // NB * NB, K):
    gat(j, 0)
    gat_wait(j, 0)
    sca(j, 0)
    sca_wait(j, 0)
  plsc.subcore_barrier()

  def wrow(t, _):
    r = sid * RPT + t * B
    pltpu.sync_copy(acc_sh.at[pl.ds(r, B)], out_hbm.at[cid, pl.ds(r, B)])
    return 0
  lax.fori_loop(0, RPT // B, wrow, 0)


def _dinv_from(degp0, degp1):
  deg = degp0 + degp1 + 1.0
  return lax.rsqrt(deg)[:, 0:1]


def _mm1_body(x_ref, w_ref, degp_ref, o_ref):
  dinv = _dinv_from(degp_ref[0], degp_ref[1])
  h = jnp.dot(x_ref[...], w_ref[0], preferred_element_type=jnp.float32)
  o_ref[0] = h * dinv


def _mm2_body(acc1_ref, hp_ref, degp_ref, w2_ref, b1_ref, o_ref):
  dinv = _dinv_from(degp_ref[0], degp_ref[1])
  acc = jnp.concatenate([acc1_ref[0] + hp_ref[0], acc1_ref[1] + hp_ref[1]],
                        axis=1)
  h1 = jnp.maximum(acc * dinv + b1_ref[...], 0.0)
  g = jnp.dot(h1 * dinv, w2_ref[...], preferred_element_type=jnp.float32)
  row = pl.program_id(0) * BLK + lax.broadcasted_iota(jnp.int32, (BLK, DPAD), 0)
  o_ref[...] = jnp.where(row < N_NODES, g, 0.0)


def _final_body(a0a1_ref, g_ref, degp_ref, b2_ref, o_ref):
  dinv = _dinv_from(degp_ref[0], degp_ref[1])
  acc = a0a1_ref[0] + a0a1_ref[1] + g_ref[...]
  o_ref[...] = acc * dinv + b2_ref[...]


def kernel(x, edge_index, W1, b1, W2, b2):
  f32 = jnp.float32
  x_pad = jnp.pad(x, ((0, NP - N_NODES), (0, 0)))
  spread = N_NODES + jnp.arange(E_PAD - E_EDGES, dtype=jnp.int32) % (NP - N_NODES)
  pad = jnp.full((E_PAD - E_EDGES,), N_NODES, jnp.int32)
  src3 = jnp.concatenate([edge_index[0], pad]).reshape(NW, K, B)
  dst3 = jnp.concatenate([edge_index[1], spread]).reshape(NW, K, B)
  spread2 = N_NODES + jnp.arange(E_PAD2 - E_EDGES, dtype=jnp.int32) % (NP - N_NODES)
  pad2 = jnp.full((E_PAD2 - E_EDGES,), N_NODES, jnp.int32)
  src2 = jnp.concatenate([edge_index[0], pad2]).reshape(NS, K2, B)
  dst2 = jnp.concatenate([edge_index[1], spread2]).reshape(NS, K2, B)
  w1_split = W1.reshape(D, NC, CW).transpose(1, 0, 2)
  w2_pad = jnp.pad(W2, ((0, 0), (0, DPAD - DOUT)))
  b1r = b1.reshape(1, D)
  b2r = jnp.pad(b2, (0, DPAD - DOUT)).reshape(1, DPAD)

  degp = _deg_kernel(dst3)

  grid = (NP // BLK,)
  degp_spec = pl.BlockSpec((2, BLK, DPAD), lambda g: (0, g, 0))
  degp_spec2 = pl.BlockSpec((2, BLK, DPAD), lambda g, c: (0, g, 0))
  hprime = pl.pallas_call(
      _mm1_body,
      grid=(NP // BLK, NC),
      in_specs=[
          pl.BlockSpec((BLK, D), lambda g, c: (g, 0)),
          pl.BlockSpec((1, D, CW), lambda g, c: (c, 0, 0)),
          degp_spec2,
      ],
      out_specs=pl.BlockSpec((1, BLK, CW), lambda g, c: (c, g, 0)),
      out_shape=jax.ShapeDtypeStruct((NC, NP, CW), f32),
  )(x_pad, w1_split, degp)

  acc1 = _agg128(hprime, src2, dst2)

  g = pl.pallas_call(
      _mm2_body,
      grid=grid,
      in_specs=[
          pl.BlockSpec((2, BLK, CW), lambda g: (0, g, 0)),
          pl.BlockSpec((2, BLK, CW), lambda g: (0, g, 0)),
          degp_spec,
          pl.BlockSpec((D, DPAD), lambda g: (0, 0)),
          pl.BlockSpec((1, D), lambda g: (0, 0)),
      ],
      out_specs=pl.BlockSpec((BLK, DPAD), lambda g: (g, 0)),
      out_shape=jax.ShapeDtypeStruct((NP, DPAD), f32),
  )(acc1, hprime, degp, w2_pad, b1r)

  acc2 = _agg16(g, src3, dst3)

  out = pl.pallas_call(
      _final_body,
      grid=grid,
      in_specs=[
          pl.BlockSpec((2, BLK, DPAD), lambda g: (0, g, 0)),
          pl.BlockSpec((BLK, DPAD), lambda g: (g, 0)),
          degp_spec,
          pl.BlockSpec((1, DPAD), lambda g: (0, 0)),
      ],
      out_specs=pl.BlockSpec((BLK, DPAD), lambda g: (g, 0)),
      out_shape=jax.ShapeDtypeStruct((NP, DPAD), f32),
  )(acc2, g, degp, b2r)

  return out[:N_NODES, :DOUT]
